# baseline (speedup 1.0000x reference)
import functools

import jax
import jax.numpy as jnp
from jax import lax
from jax.experimental import pallas as pl
from jax.experimental.pallas import tpu as pltpu
from jax.experimental.pallas import tpu_sc as plsc

N_NODES = 50000
D = 256
NUM_GRAPHS = 16
D_GRAPH = 2 * D

L = 16
NC = 2
NS = 16
NW = NC * NS
C = 80
NCHUNK = N_NODES // C
TPW = -(-NCHUNK // NW)
KV = D // L
ROWW = D + L
ACC_W = NUM_GRAPHS * ROWW


def _sc_body(hv_hbm, gid_hbm, wg_hbm, bg_hbm, part_hbm,
             hv_a, hv_b, gid_a, gid_b, wg_buf, bg_buf, acc, dotbuf,
             sem_a, sem_b):
    wid = lax.axis_index("s") * NC + lax.axis_index("c")

    zeros = jnp.zeros((L,), jnp.float32)
    for i in range(ACC_W // L):
        acc[pl.ds(L * i, L)] = zeros

    pltpu.sync_copy(wg_hbm, wg_buf)
    pltpu.sync_copy(bg_hbm, bg_buf)
    wg = [wg_buf[pl.ds(L * k, L)] for k in range(KV)]
    bgv = bg_buf[...]

    def issue(t, hv_buf, gid_buf, sem):
        c = wid + NW * t

        @pl.when(c < NCHUNK)
        def _():
            pltpu.async_copy(hv_hbm.at[pl.ds(c * C, C), :], hv_buf, sem)
            pltpu.async_copy(gid_hbm.at[pl.ds(c * C, C)],
                             gid_buf.at[pl.ds(0, C)], sem)

    def wait(t, hv_buf, gid_buf, sem):
        c = wid + NW * t

        @pl.when(c < NCHUNK)
        def _():
            pltpu.make_async_copy(
                hv_hbm.at[pl.ds(0, C), :], hv_buf, sem).wait()
            pltpu.make_async_copy(
                gid_hbm.at[pl.ds(0, C)], gid_buf.at[pl.ds(0, C)], sem).wait()

    iota16 = lax.iota(jnp.int32, L)
    idx0 = iota16 * L

    def process(t, hv_buf, gid_buf):
        c = wid + NW * t

        @pl.when(c < NCHUNK)
        def _():
            def dot_body(j, _):
                row = [hv_buf[j, pl.ds(L * k, L)] for k in range(KV)]
                p = [row[k] * wg[k] for k in range(4)]
                for k in range(4, KV):
                    p[k % 4] = p[k % 4] + row[k] * wg[k]
                dotbuf[pl.ds(j * L, L)] = (p[0] + p[1]) + (p[2] + p[3])
                return 0

            lax.fori_loop(0, C, dot_body, 0, unroll=2)

            def group_body(g, _):
                gbase = g * (L * L)
                z = plsc.load_gather(dotbuf, [idx0 + gbase])
                for l in range(1, L):
                    z = z + plsc.load_gather(dotbuf, [idx0 + (gbase + l)])
                gate = 1.0 / (1.0 + jnp.exp(-(z + bgv)))
                gidv = gid_buf[pl.ds(g * L, L)]
                uniform = gidv[0] == gidv[L - 1]

                @pl.when(uniform)
                def _():
                    gs = jnp.full((L,), gate[0], jnp.float32)
                    accv = [gs * hv_buf[g * L, pl.ds(L * k, L)]
                            for k in range(KV)]
                    cacc = gs
                    for j2 in range(1, L):
                        gs = jnp.full((L,), gate[j2], jnp.float32)
                        row = [hv_buf[g * L + j2, pl.ds(L * k, L)]
                               for k in range(KV)]
                        for k in range(KV):
                            accv[k] = accv[k] + gs * row[k]
                        cacc = cacc + gs
                    base = gidv[0] * ROWW
                    for k in range(KV):
                        plsc.addupdate(acc.at[pl.ds(base + L * k, L)],
                                       accv[k])
                    plsc.addupdate(acc.at[pl.ds(base + D, L)], cacc)

                @pl.when(jnp.logical_not(uniform))
                def _():
                    for j2 in range(L):
                        row = [hv_buf[g * L + j2, pl.ds(L * k, L)]
                               for k in range(KV)]
                        gs = jnp.full((L,), gate[j2], jnp.float32)
                        base = gidv[j2] * ROWW
                        for k in range(KV):
                            plsc.addupdate(acc.at[pl.ds(base + L * k, L)],
                                           gs * row[k])
                        plsc.addupdate(acc.at[pl.ds(base + D, L)], gs)
                return 0

            lax.fori_loop(0, C // L, group_body, 0)

    issue(0, hv_a, gid_a, sem_a)

    def pipe_body(i, _):
        ta = 2 * i
        tb = ta + 1
        wait(ta, hv_a, gid_a, sem_a)
        issue(tb, hv_b, gid_b, sem_b)
        process(ta, hv_a, gid_a)
        wait(tb, hv_b, gid_b, sem_b)

        @pl.when(tb + 1 < TPW)
        def _():
            issue(tb + 1, hv_a, gid_a, sem_a)

        process(tb, hv_b, gid_b)
        return 0

    lax.fori_loop(0, TPW // 2, pipe_body, 0)

    pltpu.sync_copy(acc, part_hbm.at[wid])


@functools.partial(
    pl.kernel,
    out_type=jax.ShapeDtypeStruct((NW, ACC_W), jnp.float32),
    mesh=plsc.VectorSubcoreMesh(core_axis_name="c", subcore_axis_name="s"),
    compiler_params=pltpu.CompilerParams(needs_layout_passes=False),
    scratch_types=[
        pltpu.VMEM((C, D), jnp.float32),
        pltpu.VMEM((C, D), jnp.float32),
        pltpu.VMEM((C + L,), jnp.int32),
        pltpu.VMEM((C + L,), jnp.int32),
        pltpu.VMEM((D,), jnp.float32),
        pltpu.VMEM((L,), jnp.float32),
        pltpu.VMEM((ACC_W,), jnp.float32),
        pltpu.VMEM((C * L,), jnp.float32),
        pltpu.SemaphoreType.DMA,
        pltpu.SemaphoreType.DMA,
    ],
)
def _sc_segment_pass(*refs):
    _sc_body(*refs)


def _combine_body(part_ref, wg_ref, bg_ref, out_ref):
    part = part_ref[...]
    s = jnp.sum(part[:, :, :D], axis=0)
    c = jnp.sum(part[:, :, D], axis=0)
    out = jnp.dot(s, wg_ref[...], preferred_element_type=jnp.float32)
    out_ref[...] = out + c[:, None] * bg_ref[...][None, :]


def kernel(hv, graph_ids, W_gate, b_gate, W_g, b_g):
    gid = graph_ids.astype(jnp.int32)
    wg_flat = W_gate.reshape(D)
    bg16 = jnp.broadcast_to(b_gate.reshape(1), (L,)).astype(jnp.float32)

    part = _sc_segment_pass(hv, gid, wg_flat, bg16)
    part3 = part.reshape(NW, NUM_GRAPHS, ROWW)

    out = pl.pallas_call(
        _combine_body,
        out_shape=jax.ShapeDtypeStruct((NUM_GRAPHS, D_GRAPH), jnp.float32),
    )(part3, W_g, b_g)
    return out


# speedup vs baseline: 1.3021x; 1.3021x over previous
import functools

import jax
import jax.numpy as jnp
from jax import lax
from jax.experimental import pallas as pl
from jax.experimental.pallas import tpu as pltpu
from jax.experimental.pallas import tpu_sc as plsc

N_NODES = 50000
D = 256
NUM_GRAPHS = 16
D_GRAPH = 2 * D

L = 16
NC = 2
NS = 16
NW = NC * NS
C = 80
NCHUNK = N_NODES // C
TPW = -(-NCHUNK // NW)
KV = D // L
ROWW = D + L
ACC_W = NUM_GRAPHS * ROWW

GBLK = 5120
NPAD = GBLK * 10


def _gate_body(hv_ref, wg_ref, bg_ref, g_ref):
    z = jnp.sum(hv_ref[...] * wg_ref[...], axis=1) + bg_ref[...]
    g_ref[pl.ds(pl.program_id(0) * GBLK, GBLK)] = 1.0 / (1.0 + jnp.exp(-z))


def _sc_body(hv_hbm, gid_hbm, g_hbm, part_hbm,
             hv_a, hv_b, gid_a, gid_b, g_a, g_b, acc, sem_a, sem_b):
    wid = lax.axis_index("s") * NC + lax.axis_index("c")

    zeros = jnp.zeros((L,), jnp.float32)
    for i in range(ACC_W // L):
        acc[pl.ds(L * i, L)] = zeros

    def issue(t, hv_buf, gid_buf, g_buf, sem):
        c = wid + NW * t

        @pl.when(c < NCHUNK)
        def _():
            pltpu.async_copy(hv_hbm.at[pl.ds(c * C, C), :], hv_buf, sem)
            pltpu.async_copy(gid_hbm.at[pl.ds(c * C, C)],
                             gid_buf.at[pl.ds(0, C)], sem)
            pltpu.async_copy(g_hbm.at[pl.ds(c * C, C)],
                             g_buf.at[pl.ds(0, C)], sem)

    def wait(t, hv_buf, gid_buf, g_buf, sem):
        c = wid + NW * t

        @pl.when(c < NCHUNK)
        def _():
            pltpu.make_async_copy(
                hv_hbm.at[pl.ds(0, C), :], hv_buf, sem).wait()
            pltpu.make_async_copy(
                gid_hbm.at[pl.ds(0, C)], gid_buf.at[pl.ds(0, C)], sem).wait()
            pltpu.make_async_copy(
                g_hbm.at[pl.ds(0, C)], g_buf.at[pl.ds(0, C)], sem).wait()

    def process(t, hv_buf, gid_buf, g_buf):
        c = wid + NW * t

        @pl.when(c < NCHUNK)
        def _():
            def group_body(g, _):
                gate = g_buf[pl.ds(g * L, L)]
                gidv = gid_buf[pl.ds(g * L, L)]
                row = [hv_buf[g * L, pl.ds(L * k, L)] for k in range(KV)]
                for j2 in range(L):
                    cur = row
                    gs = jnp.full((L,), gate[j2], jnp.float32)
                    base = gidv[j2] * ROWW
                    row = []
                    for k in range(KV):
                        if j2 + 1 < L:
                            row.append(hv_buf[g * L + j2 + 1,
                                              pl.ds(L * k, L)])
                        plsc.addupdate(acc.at[pl.ds(base + L * k, L)],
                                       gs * cur[k])
                    plsc.addupdate(acc.at[pl.ds(base + D, L)], gs)
                return 0

            lax.fori_loop(0, C // L, group_body, 0)

    issue(0, hv_a, gid_a, g_a, sem_a)

    def pipe_body(i, _):
        ta = 2 * i
        tb = ta + 1
        wait(ta, hv_a, gid_a, g_a, sem_a)
        issue(tb, hv_b, gid_b, g_b, sem_b)
        process(ta, hv_a, gid_a, g_a)
        wait(tb, hv_b, gid_b, g_b, sem_b)

        @pl.when(tb + 1 < TPW)
        def _():
            issue(tb + 1, hv_a, gid_a, g_a, sem_a)

        process(tb, hv_b, gid_b, g_b)
        return 0

    lax.fori_loop(0, TPW // 2, pipe_body, 0)

    pltpu.sync_copy(acc, part_hbm.at[wid])


@functools.partial(
    pl.kernel,
    out_type=jax.ShapeDtypeStruct((NW, ACC_W), jnp.float32),
    mesh=plsc.VectorSubcoreMesh(core_axis_name="c", subcore_axis_name="s"),
    compiler_params=pltpu.CompilerParams(needs_layout_passes=False),
    scratch_types=[
        pltpu.VMEM((C, D), jnp.float32),
        pltpu.VMEM((C, D), jnp.float32),
        pltpu.VMEM((C + L,), jnp.int32),
        pltpu.VMEM((C + L,), jnp.int32),
        pltpu.VMEM((C + L,), jnp.float32),
        pltpu.VMEM((C + L,), jnp.float32),
        pltpu.VMEM((ACC_W,), jnp.float32),
        pltpu.SemaphoreType.DMA,
        pltpu.SemaphoreType.DMA,
    ],
)
def _sc_segment_pass(*refs):
    _sc_body(*refs)


def _combine_body(part_ref, wg_ref, bg_ref, out_ref):
    part = part_ref[...]
    s = jnp.sum(part[:, :, :D], axis=0)
    c = jnp.sum(part[:, :, D], axis=0)
    out = jnp.dot(s, wg_ref[...], preferred_element_type=jnp.float32)
    out_ref[...] = out + c[:, None] * bg_ref[...][None, :]


def kernel(hv, graph_ids, W_gate, b_gate, W_g, b_g):
    gid = graph_ids.astype(jnp.int32)
    wg_row = W_gate.reshape(1, D)
    bg1 = b_gate.reshape(1).astype(jnp.float32)

    gates = pl.pallas_call(
        _gate_body,
        grid=(NPAD // GBLK,),
        in_specs=[
            pl.BlockSpec((GBLK, D), lambda i: (i, 0)),
            pl.BlockSpec((1, D), lambda i: (0, 0)),
            pl.BlockSpec((1,), lambda i: (0,)),
        ],
        out_specs=pl.BlockSpec((NPAD,), lambda i: (0,)),
        out_shape=jax.ShapeDtypeStruct((NPAD,), jnp.float32),
    )(hv, wg_row, bg1)

    part = _sc_segment_pass(hv, gid, gates)
    part3 = part.reshape(NW, NUM_GRAPHS, ROWW)

    out = pl.pallas_call(
        _combine_body,
        out_shape=jax.ShapeDtypeStruct((NUM_GRAPHS, D_GRAPH), jnp.float32),
    )(part3, W_g, b_g)
    return out


# speedup vs baseline: 1.3687x; 1.0511x over previous
import functools

import jax
import jax.numpy as jnp
from jax import lax
from jax.experimental import pallas as pl
from jax.experimental.pallas import tpu as pltpu
from jax.experimental.pallas import tpu_sc as plsc

N_NODES = 50000
D = 256
NUM_GRAPHS = 16
D_GRAPH = 2 * D

L = 16
NC = 2
NS = 16
NW = NC * NS
C = 80
NCHUNK = N_NODES // C
TPW = -(-NCHUNK // NW)
KV = D // L
ROWW = D + L
ACC_W = NUM_GRAPHS * ROWW

GBLK = 5120
GSTEPS = 5
GHALF = GBLK * GSTEPS
HALF = GHALF // C


def _gate_body(hv_ref, wg_ref, bg_ref, g_ref):
    z = jnp.sum(hv_ref[...] * wg_ref[...], axis=1) + bg_ref[...]
    g_ref[pl.ds(pl.program_id(0) * GBLK, GBLK)] = 1.0 / (1.0 + jnp.exp(-z))


def _sc_body(chunk_lo, chunk_hi, hv_hbm, gid_hbm, g_hbm, part_hbm,
             hv_a, hv_b, gid_a, gid_b, g_a, g_b, acc, sem_a, sem_b):
    tpw = -(-(chunk_hi - chunk_lo) // NW)
    wid = lax.axis_index("s") * NC + lax.axis_index("c")

    zeros = jnp.zeros((L,), jnp.float32)
    for i in range(ACC_W // L):
        acc[pl.ds(L * i, L)] = zeros

    def issue(t, hv_buf, gid_buf, g_buf, sem):
        c = chunk_lo + wid + NW * t

        @pl.when(c < chunk_hi)
        def _():
            pltpu.async_copy(hv_hbm.at[pl.ds(c * C, C), :], hv_buf, sem)
            pltpu.async_copy(gid_hbm.at[pl.ds(c * C, C)],
                             gid_buf.at[pl.ds(0, C)], sem)
            pltpu.async_copy(g_hbm.at[pl.ds((c - chunk_lo) * C, C)],
                             g_buf.at[pl.ds(0, C)], sem)

    def wait(t, hv_buf, gid_buf, g_buf, sem):
        c = chunk_lo + wid + NW * t

        @pl.when(c < chunk_hi)
        def _():
            pltpu.make_async_copy(
                hv_hbm.at[pl.ds(0, C), :], hv_buf, sem).wait()
            pltpu.make_async_copy(
                gid_hbm.at[pl.ds(0, C)], gid_buf.at[pl.ds(0, C)], sem).wait()
            pltpu.make_async_copy(
                g_hbm.at[pl.ds(0, C)], g_buf.at[pl.ds(0, C)], sem).wait()

    def process(t, hv_buf, gid_buf, g_buf):
        c = chunk_lo + wid + NW * t

        @pl.when(c < chunk_hi)
        def _():
            def group_body(g, _):
                gate = g_buf[pl.ds(g * L, L)]
                gidv = gid_buf[pl.ds(g * L, L)]
                row = [hv_buf[g * L, pl.ds(L * k, L)] for k in range(KV)]
                for j2 in range(L):
                    cur = row
                    gs = jnp.full((L,), gate[j2], jnp.float32)
                    base = gidv[j2] * ROWW
                    row = []
                    for k in range(KV):
                        if j2 + 1 < L:
                            row.append(hv_buf[g * L + j2 + 1,
                                              pl.ds(L * k, L)])
                        plsc.addupdate(acc.at[pl.ds(base + L * k, L)],
                                       gs * cur[k])
                    plsc.addupdate(acc.at[pl.ds(base + D, L)], gs)
                return 0

            lax.fori_loop(0, C // L, group_body, 0)

    issue(0, hv_a, gid_a, g_a, sem_a)

    def pipe_body(i, _):
        ta = 2 * i
        tb = ta + 1
        wait(ta, hv_a, gid_a, g_a, sem_a)
        issue(tb, hv_b, gid_b, g_b, sem_b)
        process(ta, hv_a, gid_a, g_a)
        wait(tb, hv_b, gid_b, g_b, sem_b)

        @pl.when(tb + 1 < tpw)
        def _():
            issue(tb + 1, hv_a, gid_a, g_a, sem_a)

        process(tb, hv_b, gid_b, g_b)
        return 0

    lax.fori_loop(0, (tpw + 1) // 2, pipe_body, 0)

    pltpu.sync_copy(acc, part_hbm.at[wid])


def _make_sc_pass(chunk_lo, chunk_hi):
    @functools.partial(
        pl.kernel,
        out_type=jax.ShapeDtypeStruct((NW, ACC_W), jnp.float32),
        mesh=plsc.VectorSubcoreMesh(core_axis_name="c",
                                    subcore_axis_name="s"),
        compiler_params=pltpu.CompilerParams(needs_layout_passes=False),
        scratch_types=[
            pltpu.VMEM((C, D), jnp.float32),
            pltpu.VMEM((C, D), jnp.float32),
            pltpu.VMEM((C + L,), jnp.int32),
            pltpu.VMEM((C + L,), jnp.int32),
            pltpu.VMEM((C + L,), jnp.float32),
            pltpu.VMEM((C + L,), jnp.float32),
            pltpu.VMEM((ACC_W,), jnp.float32),
            pltpu.SemaphoreType.DMA,
            pltpu.SemaphoreType.DMA,
        ],
    )
    def _pass(*refs):
        _sc_body(chunk_lo, chunk_hi, *refs)

    return _pass


_sc_pass1 = _make_sc_pass(0, HALF)
_sc_pass2 = _make_sc_pass(HALF, NCHUNK)


def _combine_body(p1_ref, p2_ref, wg_ref, bg_ref, out_ref):
    part1 = p1_ref[...]
    part2 = p2_ref[...]
    s = jnp.sum(part1[:, :, :D], axis=0) + jnp.sum(part2[:, :, :D], axis=0)
    c = jnp.sum(part1[:, :, D], axis=0) + jnp.sum(part2[:, :, D], axis=0)
    out = jnp.dot(s, wg_ref[...], preferred_element_type=jnp.float32)
    out_ref[...] = out + c[:, None] * bg_ref[...][None, :]


def _gate_pass(hv, wg_row, bg1, block_off):
    return pl.pallas_call(
        _gate_body,
        grid=(GSTEPS,),
        in_specs=[
            pl.BlockSpec((GBLK, D), lambda i: (i + block_off, 0)),
            pl.BlockSpec((1, D), lambda i: (0, 0)),
            pl.BlockSpec((1,), lambda i: (0,)),
        ],
        out_specs=pl.BlockSpec((GHALF,), lambda i: (0,)),
        out_shape=jax.ShapeDtypeStruct((GHALF,), jnp.float32),
    )(hv, wg_row, bg1)


def kernel(hv, graph_ids, W_gate, b_gate, W_g, b_g):
    gid = graph_ids.astype(jnp.int32)
    wg_row = W_gate.reshape(1, D)
    bg1 = b_gate.reshape(1).astype(jnp.float32)

    g1 = _gate_pass(hv, wg_row, bg1, 0)
    p1 = _sc_pass1(hv, gid, g1)
    g2 = _gate_pass(hv, wg_row, bg1, GSTEPS)
    p2 = _sc_pass2(hv, gid, g2)

    out = pl.pallas_call(
        _combine_body,
        out_shape=jax.ShapeDtypeStruct((NUM_GRAPHS, D_GRAPH), jnp.float32),
    )(p1.reshape(NW, NUM_GRAPHS, ROWW), p2.reshape(NW, NUM_GRAPHS, ROWW),
      W_g, b_g)
    return out


# speedup vs baseline: 1.5250x; 1.1142x over previous
import functools

import jax
import jax.numpy as jnp
from jax import lax
from jax.experimental import pallas as pl
from jax.experimental.pallas import tpu as pltpu
from jax.experimental.pallas import tpu_sc as plsc

N_NODES = 50000
D = 256
NUM_GRAPHS = 16
D_GRAPH = 2 * D

L = 16
NC = 2
NS = 16
NW = NC * NS
C = 80
NCHUNK = N_NODES // C
TPW = -(-NCHUNK // NW)
KV = D // L
ROWW = D + L
ACC_W = NUM_GRAPHS * ROWW

GBLK = 5120
GSTEPS = 5
GHALF = GBLK * GSTEPS
HALF = GHALF // C


def _gate_body(hv_ref, wg_ref, bg_ref, g_ref):
    hv3 = hv_ref[...].reshape(GBLK // 128, 128, D)
    z = jnp.sum(hv3 * wg_ref[...].reshape(1, 1, D), axis=2)
    g_ref[...] = 1.0 / (1.0 + jnp.exp(-(z + bg_ref[...].reshape(1, 1))))


def _sc_body(chunk_lo, chunk_hi, hv_hbm, gid_hbm, g_hbm, part_hbm,
             hv_a, hv_b, gid_a, gid_b, g_a, g_b, acc, sem_a, sem_b):
    tpw = -(-(chunk_hi - chunk_lo) // NW)
    wid = lax.axis_index("s") * NC + lax.axis_index("c")

    zeros = jnp.zeros((L,), jnp.float32)
    for i in range(ACC_W // L):
        acc[pl.ds(L * i, L)] = zeros

    def issue(t, hv_buf, gid_buf, g_buf, sem):
        c = chunk_lo + wid + NW * t

        @pl.when(c < chunk_hi)
        def _():
            pltpu.async_copy(hv_hbm.at[pl.ds(c * C, C), :], hv_buf, sem)
            pltpu.async_copy(gid_hbm.at[pl.ds(c * C, C)],
                             gid_buf.at[pl.ds(0, C)], sem)
            pltpu.async_copy(g_hbm.at[pl.ds((c - chunk_lo) * C, C)],
                             g_buf.at[pl.ds(0, C)], sem)

    def wait(t, hv_buf, gid_buf, g_buf, sem):
        c = chunk_lo + wid + NW * t

        @pl.when(c < chunk_hi)
        def _():
            pltpu.make_async_copy(
                hv_hbm.at[pl.ds(0, C), :], hv_buf, sem).wait()
            pltpu.make_async_copy(
                gid_hbm.at[pl.ds(0, C)], gid_buf.at[pl.ds(0, C)], sem).wait()
            pltpu.make_async_copy(
                g_hbm.at[pl.ds(0, C)], g_buf.at[pl.ds(0, C)], sem).wait()

    def process(t, hv_buf, gid_buf, g_buf):
        c = chunk_lo + wid + NW * t

        @pl.when(c < chunk_hi)
        def _():
            def group_body(g, _):
                gate = g_buf[pl.ds(g * L, L)]
                gidv = gid_buf[pl.ds(g * L, L)]
                row = [hv_buf[g * L, pl.ds(L * k, L)] for k in range(KV)]
                for j2 in range(L):
                    cur = row
                    gs = jnp.full((L,), gate[j2], jnp.float32)
                    base = gidv[j2] * ROWW
                    row = []
                    for k in range(KV):
                        if j2 + 1 < L:
                            row.append(hv_buf[g * L + j2 + 1,
                                              pl.ds(L * k, L)])
                        plsc.addupdate(acc.at[pl.ds(base + L * k, L)],
                                       gs * cur[k])
                    plsc.addupdate(acc.at[pl.ds(base + D, L)], gs)
                return 0

            lax.fori_loop(0, C // L, group_body, 0)

    issue(0, hv_a, gid_a, g_a, sem_a)

    def pipe_body(i, _):
        ta = 2 * i
        tb = ta + 1
        wait(ta, hv_a, gid_a, g_a, sem_a)
        issue(tb, hv_b, gid_b, g_b, sem_b)
        process(ta, hv_a, gid_a, g_a)
        wait(tb, hv_b, gid_b, g_b, sem_b)

        @pl.when(tb + 1 < tpw)
        def _():
            issue(tb + 1, hv_a, gid_a, g_a, sem_a)

        process(tb, hv_b, gid_b, g_b)
        return 0

    lax.fori_loop(0, (tpw + 1) // 2, pipe_body, 0)

    pltpu.sync_copy(acc, part_hbm.at[wid])


def _make_sc_pass(chunk_lo, chunk_hi):
    @functools.partial(
        pl.kernel,
        out_type=jax.ShapeDtypeStruct((NW, ACC_W), jnp.float32),
        mesh=plsc.VectorSubcoreMesh(core_axis_name="c",
                                    subcore_axis_name="s"),
        compiler_params=pltpu.CompilerParams(needs_layout_passes=False),
        scratch_types=[
            pltpu.VMEM((C, D), jnp.float32),
            pltpu.VMEM((C, D), jnp.float32),
            pltpu.VMEM((C + L,), jnp.int32),
            pltpu.VMEM((C + L,), jnp.int32),
            pltpu.VMEM((C + L,), jnp.float32),
            pltpu.VMEM((C + L,), jnp.float32),
            pltpu.VMEM((ACC_W,), jnp.float32),
            pltpu.SemaphoreType.DMA,
            pltpu.SemaphoreType.DMA,
        ],
    )
    def _pass(*refs):
        _sc_body(chunk_lo, chunk_hi, *refs)

    return _pass


_sc_pass1 = _make_sc_pass(0, HALF)
_sc_pass2 = _make_sc_pass(HALF, NCHUNK)


def _combine_body(p1_ref, p2_ref, wg_ref, bg_ref, out_ref):
    part1 = p1_ref[...]
    part2 = p2_ref[...]
    s = jnp.sum(part1[:, :, :D], axis=0) + jnp.sum(part2[:, :, :D], axis=0)
    c = jnp.sum(part1[:, :, D], axis=0) + jnp.sum(part2[:, :, D], axis=0)
    out = jnp.dot(s, wg_ref[...], preferred_element_type=jnp.float32)
    out_ref[...] = out + c[:, None] * bg_ref[...][None, :]


def _gate_pass(hv, wg_row, bg1, block_off):
    return pl.pallas_call(
        _gate_body,
        grid=(GSTEPS,),
        in_specs=[
            pl.BlockSpec((GBLK, D), lambda i: (i + block_off, 0)),
            pl.BlockSpec((1, D), lambda i: (0, 0)),
            pl.BlockSpec((1,), lambda i: (0,)),
        ],
        out_specs=pl.BlockSpec((GBLK // 128, 128), lambda i: (i, 0)),
        out_shape=jax.ShapeDtypeStruct((GHALF // 128, 128), jnp.float32),
    )(hv, wg_row, bg1)


def kernel(hv, graph_ids, W_gate, b_gate, W_g, b_g):
    gid = graph_ids.astype(jnp.int32)
    wg_row = W_gate.reshape(1, D)
    bg1 = b_gate.reshape(1).astype(jnp.float32)

    g1 = _gate_pass(hv, wg_row, bg1, 0).reshape(GHALF)
    p1 = _sc_pass1(hv, gid, g1)
    g2 = _gate_pass(hv, wg_row, bg1, GSTEPS).reshape(GHALF)
    p2 = _sc_pass2(hv, gid, g2)

    out = pl.pallas_call(
        _combine_body,
        out_shape=jax.ShapeDtypeStruct((NUM_GRAPHS, D_GRAPH), jnp.float32),
    )(p1.reshape(NW, NUM_GRAPHS, ROWW), p2.reshape(NW, NUM_GRAPHS, ROWW),
      W_g, b_g)
    return out


# speedup vs baseline: 1.6084x; 1.0547x over previous
import functools

import jax
import jax.numpy as jnp
from jax import lax
from jax.experimental import pallas as pl
from jax.experimental.pallas import tpu as pltpu
from jax.experimental.pallas import tpu_sc as plsc

N_NODES = 50000
D = 256
NUM_GRAPHS = 16
D_GRAPH = 2 * D

L = 16
NC = 2
NS = 16
NW = NC * NS
C = 80
NCHUNK = N_NODES // C
TPW = -(-NCHUNK // NW)
KV = D // L
ROWW = D + L
ACC_W = NUM_GRAPHS * ROWW

GBLK = 5120
GSTEPS = 10
GFULL = GBLK * GSTEPS


def _gate_body(hv_ref, wg_ref, bg_ref, g_ref):
    hv3 = hv_ref[...].reshape(GBLK // 128, 128, D)
    z = jnp.sum(hv3 * wg_ref[...].reshape(1, 1, D), axis=2)
    g_ref[...] = 1.0 / (1.0 + jnp.exp(-(z + bg_ref[...].reshape(1, 1))))


def _sc_body(chunk_lo, chunk_hi, hv_hbm, gid_hbm, g_hbm, part_hbm,
             hv_a, hv_b, gid_a, gid_b, g_a, g_b, acc, sem_a, sem_b):
    tpw = -(-(chunk_hi - chunk_lo) // NW)
    wid = lax.axis_index("s") * NC + lax.axis_index("c")

    zeros = jnp.zeros((L,), jnp.float32)
    for i in range(ACC_W // L):
        acc[pl.ds(L * i, L)] = zeros

    def issue(t, hv_buf, gid_buf, g_buf, sem):
        c = chunk_lo + wid + NW * t

        @pl.when(c < chunk_hi)
        def _():
            pltpu.async_copy(hv_hbm.at[pl.ds(c * C, C), :], hv_buf, sem)
            pltpu.async_copy(gid_hbm.at[pl.ds(c * C, C)],
                             gid_buf.at[pl.ds(0, C)], sem)
            pltpu.async_copy(g_hbm.at[pl.ds((c - chunk_lo) * C, C)],
                             g_buf.at[pl.ds(0, C)], sem)

    def wait(t, hv_buf, gid_buf, g_buf, sem):
        c = chunk_lo + wid + NW * t

        @pl.when(c < chunk_hi)
        def _():
            pltpu.make_async_copy(
                hv_hbm.at[pl.ds(0, C), :], hv_buf, sem).wait()
            pltpu.make_async_copy(
                gid_hbm.at[pl.ds(0, C)], gid_buf.at[pl.ds(0, C)], sem).wait()
            pltpu.make_async_copy(
                g_hbm.at[pl.ds(0, C)], g_buf.at[pl.ds(0, C)], sem).wait()

    def process(t, hv_buf, gid_buf, g_buf):
        c = chunk_lo + wid + NW * t

        @pl.when(c < chunk_hi)
        def _():
            def group_body(g, _):
                gate = g_buf[pl.ds(g * L, L)]
                gidv = gid_buf[pl.ds(g * L, L)]
                row = [hv_buf[g * L, pl.ds(L * k, L)] for k in range(KV)]
                for j2 in range(L):
                    cur = row
                    gs = jnp.full((L,), gate[j2], jnp.float32)
                    base = gidv[j2] * ROWW
                    row = []
                    for k in range(KV):
                        if j2 + 1 < L:
                            row.append(hv_buf[g * L + j2 + 1,
                                              pl.ds(L * k, L)])
                        plsc.addupdate(acc.at[pl.ds(base + L * k, L)],
                                       gs * cur[k])
                    plsc.addupdate(acc.at[pl.ds(base + D, L)], gs)
                return 0

            lax.fori_loop(0, C // L, group_body, 0)

    issue(0, hv_a, gid_a, g_a, sem_a)

    def pipe_body(i, _):
        ta = 2 * i
        tb = ta + 1
        wait(ta, hv_a, gid_a, g_a, sem_a)
        issue(tb, hv_b, gid_b, g_b, sem_b)
        process(ta, hv_a, gid_a, g_a)
        wait(tb, hv_b, gid_b, g_b, sem_b)

        @pl.when(tb + 1 < tpw)
        def _():
            issue(tb + 1, hv_a, gid_a, g_a, sem_a)

        process(tb, hv_b, gid_b, g_b)
        return 0

    lax.fori_loop(0, (tpw + 1) // 2, pipe_body, 0)

    pltpu.sync_copy(acc, part_hbm.at[wid])


def _make_sc_pass(chunk_lo, chunk_hi):
    @functools.partial(
        pl.kernel,
        out_type=jax.ShapeDtypeStruct((NW, ACC_W), jnp.float32),
        mesh=plsc.VectorSubcoreMesh(core_axis_name="c",
                                    subcore_axis_name="s"),
        compiler_params=pltpu.CompilerParams(needs_layout_passes=False),
        scratch_types=[
            pltpu.VMEM((C, D), jnp.float32),
            pltpu.VMEM((C, D), jnp.float32),
            pltpu.VMEM((C + L,), jnp.int32),
            pltpu.VMEM((C + L,), jnp.int32),
            pltpu.VMEM((C + L,), jnp.float32),
            pltpu.VMEM((C + L,), jnp.float32),
            pltpu.VMEM((ACC_W,), jnp.float32),
            pltpu.SemaphoreType.DMA,
            pltpu.SemaphoreType.DMA,
        ],
    )
    def _pass(*refs):
        _sc_body(chunk_lo, chunk_hi, *refs)

    return _pass


_sc_pass = _make_sc_pass(0, NCHUNK)


def _combine_body(part_ref, wg_ref, bg_ref, out_ref):
    part = part_ref[...]
    s = jnp.sum(part[:, :, :D], axis=0)
    c = jnp.sum(part[:, :, D], axis=0)
    out = jnp.dot(s, wg_ref[...], preferred_element_type=jnp.float32)
    out_ref[...] = out + c[:, None] * bg_ref[...][None, :]


def kernel(hv, graph_ids, W_gate, b_gate, W_g, b_g):
    gid = graph_ids.astype(jnp.int32)
    wg_row = W_gate.reshape(1, D)
    bg1 = b_gate.reshape(1).astype(jnp.float32)

    gates = pl.pallas_call(
        _gate_body,
        grid=(GSTEPS,),
        in_specs=[
            pl.BlockSpec((GBLK, D), lambda i: (i, 0)),
            pl.BlockSpec((1, D), lambda i: (0, 0)),
            pl.BlockSpec((1,), lambda i: (0,)),
        ],
        out_specs=pl.BlockSpec((GBLK // 128, 128), lambda i: (i, 0)),
        out_shape=jax.ShapeDtypeStruct((GFULL // 128, 128), jnp.float32),
    )(hv, wg_row, bg1).reshape(GFULL)

    part = _sc_pass(hv, gid, gates)

    out = pl.pallas_call(
        _combine_body,
        out_shape=jax.ShapeDtypeStruct((NUM_GRAPHS, D_GRAPH), jnp.float32),
    )(part.reshape(NW, NUM_GRAPHS, ROWW), W_g, b_g)
    return out
